# Initial kernel scaffold; baseline (speedup 1.0000x reference)
import jax
import jax.numpy as jnp
from jax.experimental import pallas as pl


def kernel(x, edge_index, W1, b1, W2, b2, Wc, bc):
    raise NotImplementedError("write your pallas kernel here")


# speedup vs baseline: 17.4010x; 17.4010x over previous
import functools

import jax
import jax.numpy as jnp
from jax import lax
from jax.experimental import pallas as pl
from jax.experimental.pallas import tpu as pltpu
from jax.experimental.pallas import tpu_sc as plsc

N = 10000
E = 160000
D_IN = 256
D_H = 64
D_OUT = 512
D_CLS = 40

NC = 2
NS = 16
NW = NC * NS
C = 125
EPW = E // NW
NCHUNK = EPW // C
NPAD = 10240
RPT = NPAD // NS
DEGW = 16
HW = 128

_MESH = plsc.VectorSubcoreMesh(
    core_axis_name="c", subcore_axis_name="s", num_cores=NC, num_subcores=NS)


def _wid():
    return lax.axis_index("s") * NC + lax.axis_index("c")


def _deg_body(dst2, ones_h, zeros_h, out, idx_v, ones_v, acc_sh, sem):
    c = lax.axis_index("c")
    s = lax.axis_index("s")
    w = _wid()

    @pl.when(s == 0)
    def _():
        pltpu.sync_copy(zeros_h, acc_sh)

    pltpu.sync_copy(ones_h, ones_v)
    plsc.subcore_barrier()

    def step(j, carry):
        pltpu.sync_copy(dst2.at[w * NCHUNK + j], idx_v)
        pltpu.sync_copy(ones_v, acc_sh.at[idx_v], add=True)
        return carry

    lax.fori_loop(0, NCHUNK, step, 0)
    plsc.subcore_barrier()
    pltpu.sync_copy(acc_sh.at[pl.ds(s * RPT, RPT)],
                    out.at[c, pl.ds(s * RPT, RPT)])


_deg_call = pl.kernel(
    _deg_body,
    out_type=jax.ShapeDtypeStruct((NC, NPAD, DEGW), jnp.float32),
    mesh=_MESH,
    scratch_types=[
        pltpu.VMEM((C,), jnp.int32),
        pltpu.VMEM((C, DEGW), jnp.float32),
        pltpu.VMEM_SHARED((NPAD, DEGW), jnp.float32),
        pltpu.SemaphoreType.DMA,
    ],
)


def _prop_body(zn, src2, dst2, zeros_h, out, idx_s, idx_d, rows, acc_sh, sem):
    c = lax.axis_index("c")
    s = lax.axis_index("s")
    w = _wid()

    @pl.when(s == 0)
    def _():
        pltpu.sync_copy(zeros_h, acc_sh)

    plsc.subcore_barrier()

    def step(j, carry):
        r = w * NCHUNK + j
        pltpu.sync_copy(src2.at[r], idx_s)
        pltpu.sync_copy(dst2.at[r], idx_d)
        pltpu.async_copy(zn.at[idx_s], rows, sem).wait()
        pltpu.sync_copy(rows, acc_sh.at[idx_d], add=True)
        return carry

    lax.fori_loop(0, NCHUNK, step, 0)
    plsc.subcore_barrier()
    pltpu.sync_copy(acc_sh.at[pl.ds(s * RPT, RPT)],
                    out.at[c, pl.ds(s * RPT, RPT)])


_prop_call = pl.kernel(
    _prop_body,
    out_type=jax.ShapeDtypeStruct((NC, NPAD, HW), jnp.float32),
    mesh=_MESH,
    scratch_types=[
        pltpu.VMEM((C,), jnp.int32),
        pltpu.VMEM((C,), jnp.int32),
        pltpu.VMEM((C, HW), jnp.float32),
        pltpu.VMEM_SHARED((NPAD, HW), jnp.float32),
        pltpu.SemaphoreType.DMA,
    ],
)


_R = 1000


def _dinv(degp_ref):
    deg = degp_ref[0] + degp_ref[1] + 1.0
    return lax.rsqrt(deg[:, 0:1])


def _k1_body(x_ref, w1_ref, degp_ref, zn1_ref):
    m = jnp.dot(x_ref[...], w1_ref[...], preferred_element_type=jnp.float32)
    zn1_ref[...] = jnp.concatenate(
        [m * _dinv(degp_ref), jnp.zeros((_R, HW - D_H), jnp.float32)], axis=1)


def _k2_body(accp_ref, zn1_ref, degp_ref, b1_ref, zn2_ref):
    dinv = _dinv(degp_ref)
    agg = (accp_ref[0] + accp_ref[1] + zn1_ref[...])[:, :D_H]
    t = dinv * agg + b1_ref[...]
    h1 = jnp.where(t >= 0.0, t, 0.01 * t)
    zn2_ref[...] = jnp.concatenate(
        [dinv * h1, jnp.zeros((_R, HW - D_H), jnp.float32)], axis=1)


def _k3_body(accp_ref, zn2_ref, degp_ref, w2_ref, b2_ref, wc_ref, bc_ref,
             h2_ref, pred_ref):
    dinv = _dinv(degp_ref)
    out2 = dinv * (accp_ref[0] + accp_ref[1] + zn2_ref[...])[:, :D_H]
    t = jnp.dot(out2, w2_ref[...], preferred_element_type=jnp.float32) \
        + b2_ref[...]
    h2 = jnp.where(t >= 0.0, t, 0.01 * t)
    h2_ref[...] = h2
    logits = lax.dot_general(
        h2, wc_ref[...], (((1,), (1,)), ((), ())),
        preferred_element_type=jnp.float32) + bc_ref[...]
    mx = jnp.max(logits, axis=1, keepdims=True)
    e = jnp.exp(logits - mx)
    pred_ref[...] = e / jnp.sum(e, axis=1, keepdims=True)


def _row_blk(minor):
    return pl.BlockSpec((_R, minor), lambda i: (i, 0))


def _pair_blk(minor):
    return pl.BlockSpec((2, _R, minor), lambda i: (0, i, 0))


def _full(shape):
    return pl.BlockSpec(shape, lambda i: tuple(0 for _ in shape))


_GRID = N // _R

_k1_call = pl.pallas_call(
    _k1_body,
    grid=(_GRID,),
    in_specs=[_row_blk(D_IN), _full((D_IN, D_H)), _pair_blk(DEGW)],
    out_specs=_row_blk(HW),
    out_shape=jax.ShapeDtypeStruct((N, HW), jnp.float32),
)

_k2_call = pl.pallas_call(
    _k2_body,
    grid=(_GRID,),
    in_specs=[_pair_blk(HW), _row_blk(HW), _pair_blk(DEGW),
              _full((1, D_H))],
    out_specs=_row_blk(HW),
    out_shape=jax.ShapeDtypeStruct((N, HW), jnp.float32),
)

_k3_call = pl.pallas_call(
    _k3_body,
    grid=(_GRID,),
    in_specs=[_pair_blk(HW), _row_blk(HW), _pair_blk(DEGW),
              _full((D_H, D_OUT)), _full((1, D_OUT)),
              _full((D_CLS, D_OUT)), _full((1, D_CLS))],
    out_specs=[_row_blk(D_OUT), _row_blk(D_CLS)],
    out_shape=[jax.ShapeDtypeStruct((N, D_OUT), jnp.float32),
               jax.ShapeDtypeStruct((N, D_CLS), jnp.float32)],
)


def kernel(x, edge_index, W1, b1, W2, b2, Wc, bc):
    src2 = edge_index[0].reshape(NW * NCHUNK, C)
    dst2 = edge_index[1].reshape(NW * NCHUNK, C)
    ones_h = jnp.ones((C, DEGW), jnp.float32)
    zeros_d = jnp.zeros((NPAD, DEGW), jnp.float32)
    zeros_h = jnp.zeros((NPAD, HW), jnp.float32)

    degp = _deg_call(dst2, ones_h, zeros_d)
    zn1 = _k1_call(x, W1, degp)
    acc1 = _prop_call(zn1, src2, dst2, zeros_h)
    zn2 = _k2_call(acc1, zn1, degp, b1.reshape(1, D_H))
    acc2 = _prop_call(zn2, src2, dst2, zeros_h)
    h2, pred = _k3_call(acc2, zn2, degp, W2, b2.reshape(1, D_OUT),
                        Wc, bc.reshape(1, D_CLS))
    return (h2, pred)


# speedup vs baseline: 29.2310x; 1.6798x over previous
import functools

import jax
import jax.numpy as jnp
from jax import lax
from jax.experimental import pallas as pl
from jax.experimental.pallas import tpu as pltpu
from jax.experimental.pallas import tpu_sc as plsc

N = 10000
E = 160000
D_IN = 256
D_H = 64
D_OUT = 512
D_CLS = 40

NC = 2
NS = 16
NW = NC * NS
C = 125
EPW = E // NW
NCHUNK = EPW // C
NPAD = 10240
RPT = NPAD // NS
DEGW = 16
HW = 128

_MESH = plsc.VectorSubcoreMesh(
    core_axis_name="c", subcore_axis_name="s", num_cores=NC, num_subcores=NS)


def _wid():
    return lax.axis_index("s") * NC + lax.axis_index("c")


NBUF = 2
NGRP = NCHUNK // NBUF


def _deg_body(dst2, ones_h, zeros_h, out, idxd_v, ones_v, acc_sh, sem):
    c = lax.axis_index("c")
    s = lax.axis_index("s")
    w = _wid()

    pltpu.sync_copy(zeros_h.at[pl.ds(s * RPT, RPT)],
                    acc_sh.at[pl.ds(s * RPT, RPT)])
    pltpu.sync_copy(dst2.at[pl.ds(w * NCHUNK, NCHUNK)], idxd_v)
    pltpu.sync_copy(ones_h, ones_v)
    plsc.subcore_barrier()

    def step(j, carry):
        pltpu.sync_copy(ones_v, acc_sh.at[idxd_v.at[j]], add=True)
        return carry

    lax.fori_loop(0, NCHUNK, step, 0)
    plsc.subcore_barrier()
    pltpu.sync_copy(acc_sh.at[pl.ds(s * RPT, RPT)],
                    out.at[c, pl.ds(s * RPT, RPT)])


_deg_call = pl.kernel(
    _deg_body,
    out_type=jax.ShapeDtypeStruct((NC, NPAD, DEGW), jnp.float32),
    mesh=_MESH,
    scratch_types=[
        pltpu.VMEM((NCHUNK, C), jnp.int32),
        pltpu.VMEM((C, DEGW), jnp.float32),
        pltpu.VMEM_SHARED((NPAD, DEGW), jnp.float32),
        pltpu.SemaphoreType.DMA,
    ],
)


def _prop_body(zn, src2, dst2, zeros_h, out, idxs_v, idxd_v, rows, acc_sh,
               *sems):
    c = lax.axis_index("c")
    s = lax.axis_index("s")
    w = _wid()

    pltpu.sync_copy(zeros_h.at[pl.ds(s * RPT, RPT)],
                    acc_sh.at[pl.ds(s * RPT, RPT)])
    pltpu.sync_copy(src2.at[pl.ds(w * NCHUNK, NCHUNK)], idxs_v)
    pltpu.sync_copy(dst2.at[pl.ds(w * NCHUNK, NCHUNK)], idxd_v)
    plsc.subcore_barrier()

    def fire(b, j):
        pltpu.async_copy(zn.at[idxs_v.at[j]], rows.at[b], sems[b])

    def wait(b, j):
        pltpu.make_async_copy(zn.at[idxs_v.at[j]], rows.at[b],
                              sems[b]).wait()

    for b in range(NBUF):
        fire(b, b)

    def group(g, carry):
        for b in range(NBUF):
            j = g * NBUF + b
            wait(b, j)
            pltpu.sync_copy(rows.at[b], acc_sh.at[idxd_v.at[j]], add=True)

            @pl.when(j + NBUF < NCHUNK)
            def _():
                fire(b, j + NBUF)
        return carry

    lax.fori_loop(0, NGRP, group, 0)
    plsc.subcore_barrier()
    pltpu.sync_copy(acc_sh.at[pl.ds(s * RPT, RPT)],
                    out.at[c, pl.ds(s * RPT, RPT)])


_prop_call = pl.kernel(
    _prop_body,
    out_type=jax.ShapeDtypeStruct((NC, NPAD, HW), jnp.float32),
    mesh=_MESH,
    scratch_types=[
        pltpu.VMEM((NCHUNK, C), jnp.int32),
        pltpu.VMEM((NCHUNK, C), jnp.int32),
        pltpu.VMEM((NBUF, C, HW), jnp.float32),
        pltpu.VMEM_SHARED((NPAD, HW), jnp.float32),
    ] + [pltpu.SemaphoreType.DMA] * NBUF,
)


_R = 1000


def _dinv(degp_ref):
    deg = degp_ref[0] + degp_ref[1] + 1.0
    return lax.rsqrt(deg[:, 0:1])


def _k1_body(x_ref, w1_ref, degp_ref, zn1_ref):
    m = jnp.dot(x_ref[...], w1_ref[...], preferred_element_type=jnp.float32)
    zn1_ref[...] = jnp.concatenate(
        [m * _dinv(degp_ref), jnp.zeros((_R, HW - D_H), jnp.float32)], axis=1)


def _k2_body(accp_ref, zn1_ref, degp_ref, b1_ref, zn2_ref):
    dinv = _dinv(degp_ref)
    agg = (accp_ref[0] + accp_ref[1] + zn1_ref[...])[:, :D_H]
    t = dinv * agg + b1_ref[...]
    h1 = jnp.where(t >= 0.0, t, 0.01 * t)
    zn2_ref[...] = jnp.concatenate(
        [dinv * h1, jnp.zeros((_R, HW - D_H), jnp.float32)], axis=1)


def _k3_body(accp_ref, zn2_ref, degp_ref, w2_ref, b2_ref, wc_ref, bc_ref,
             h2_ref, pred_ref):
    dinv = _dinv(degp_ref)
    out2 = dinv * (accp_ref[0] + accp_ref[1] + zn2_ref[...])[:, :D_H]
    t = jnp.dot(out2, w2_ref[...], preferred_element_type=jnp.float32) \
        + b2_ref[...]
    h2 = jnp.where(t >= 0.0, t, 0.01 * t)
    h2_ref[...] = h2
    logits = lax.dot_general(
        h2, wc_ref[...], (((1,), (1,)), ((), ())),
        preferred_element_type=jnp.float32) + bc_ref[...]
    mx = jnp.max(logits, axis=1, keepdims=True)
    e = jnp.exp(logits - mx)
    pred_ref[...] = e / jnp.sum(e, axis=1, keepdims=True)


def _row_blk(minor):
    return pl.BlockSpec((_R, minor), lambda i: (i, 0))


def _pair_blk(minor):
    return pl.BlockSpec((2, _R, minor), lambda i: (0, i, 0))


def _full(shape):
    return pl.BlockSpec(shape, lambda i: tuple(0 for _ in shape))


_GRID = N // _R

_k1_call = pl.pallas_call(
    _k1_body,
    grid=(_GRID,),
    in_specs=[_row_blk(D_IN), _full((D_IN, D_H)), _pair_blk(DEGW)],
    out_specs=_row_blk(HW),
    out_shape=jax.ShapeDtypeStruct((N, HW), jnp.float32),
)

_k2_call = pl.pallas_call(
    _k2_body,
    grid=(_GRID,),
    in_specs=[_pair_blk(HW), _row_blk(HW), _pair_blk(DEGW),
              _full((1, D_H))],
    out_specs=_row_blk(HW),
    out_shape=jax.ShapeDtypeStruct((N, HW), jnp.float32),
)

_k3_call = pl.pallas_call(
    _k3_body,
    grid=(_GRID,),
    in_specs=[_pair_blk(HW), _row_blk(HW), _pair_blk(DEGW),
              _full((D_H, D_OUT)), _full((1, D_OUT)),
              _full((D_CLS, D_OUT)), _full((1, D_CLS))],
    out_specs=[_row_blk(D_OUT), _row_blk(D_CLS)],
    out_shape=[jax.ShapeDtypeStruct((N, D_OUT), jnp.float32),
               jax.ShapeDtypeStruct((N, D_CLS), jnp.float32)],
)


def kernel(x, edge_index, W1, b1, W2, b2, Wc, bc):
    src2 = edge_index[0].reshape(NW * NCHUNK, C)
    dst2 = edge_index[1].reshape(NW * NCHUNK, C)
    ones_h = jnp.ones((C, DEGW), jnp.float32)
    zeros_d = jnp.zeros((NPAD, DEGW), jnp.float32)
    zeros_h = jnp.zeros((NPAD, HW), jnp.float32)

    degp = _deg_call(dst2, ones_h, zeros_d)
    zn1 = _k1_call(x, W1, degp)
    acc1 = _prop_call(zn1, src2, dst2, zeros_h)
    zn2 = _k2_call(acc1, zn1, degp, b1.reshape(1, D_H))
    acc2 = _prop_call(zn2, src2, dst2, zeros_h)
    h2, pred = _k3_call(acc2, zn2, degp, W2, b2.reshape(1, D_OUT),
                        Wc, bc.reshape(1, D_CLS))
    return (h2, pred)
